# baseline (speedup 1.0000x reference)
import functools

import jax
import jax.numpy as jnp
from jax import lax
from jax.experimental import pallas as pl
from jax.experimental.pallas import tpu as pltpu
from jax.experimental.pallas import tpu_sc as plsc

N = 10000
E = 320000
D = 128
H = 128
C = 10
G = 16

NC = 2
NS = 16
NW = NC * NS

NP = 10240
TW = NP // NS
K = 128
EP = 327680
EWP = EP // NW
ECP = EP // NC
NCHUNK = EWP // K

BLK = 1024
GRID = NP // BLK

_mesh = plsc.VectorSubcoreMesh(core_axis_name="c", subcore_axis_name="s",
                               num_cores=NC, num_subcores=NS)


@functools.partial(
    pl.kernel,
    out_type=jax.ShapeDtypeStruct((NC, NP, H), jnp.float32),
    mesh=_mesh,
    scratch_types=[
        pltpu.VMEM((EWP,), jnp.int32),
        pltpu.VMEM((K,), jnp.int32),
        pltpu.VMEM((2, K, H), jnp.float32),
        pltpu.VMEM_SHARED((NP, H), jnp.float32),
        pltpu.SemaphoreType.DMA,
        pltpu.SemaphoreType.DMA,
    ],
)
def _sc_scatter(hs_hbm, src_hbm, dst_hbm, out_hbm, src_v, dstk_v, rows_v,
                acc_sh, sem0, sem1):
    c = lax.axis_index("c")
    s = lax.axis_index("s")

    base0 = c * ECP + s * EWP
    pltpu.sync_copy(src_hbm.at[pl.ds(base0, EWP)], src_v)

    z = rows_v.at[0]

    def _zero_row(r, _):
        for l in range(H // 16):
            z[r, pl.ds(l * 16, 16)] = jnp.zeros((16,), jnp.float32)
        return _
    lax.fori_loop(0, K, _zero_row, None)

    def _zero_slice(j, _):
        pltpu.sync_copy(z, acc_sh.at[pl.ds(s * TW + j * K, K)])
        return _
    lax.fori_loop(0, TW // K, _zero_slice, None)
    plsc.subcore_barrier()

    def _issue(chunk, b, sem):
        pltpu.async_copy(hs_hbm.at[src_v.at[pl.ds(chunk * K, K)]], rows_v.at[b], sem)

    def _drain(b, sem):
        pltpu.make_async_copy(hs_hbm.at[pl.ds(0, K)], rows_v.at[b], sem).wait()

    def _scatter(chunk, b):
        pltpu.sync_copy(dst_hbm.at[pl.ds(base0 + chunk * K, K)], dstk_v)
        pltpu.sync_copy(rows_v.at[b], acc_sh.at[dstk_v], add=True)

    _issue(0, 0, sem0)

    def _edge_pair(g, _):
        c0 = 2 * g
        _issue(c0 + 1, 1, sem1)
        _drain(0, sem0)
        _scatter(c0, 0)
        nxt = c0 + 2
        nxt = jnp.where(nxt == NCHUNK, 0, nxt)
        _issue(nxt, 0, sem0)
        _drain(1, sem1)
        _scatter(c0 + 1, 1)
        return _
    lax.fori_loop(0, NCHUNK // 2, _edge_pair, None)
    _drain(0, sem0)

    plsc.subcore_barrier()

    def _writeback(j, _):
        r0 = s * TW + j * K
        pltpu.sync_copy(acc_sh.at[pl.ds(r0, K)], z)
        pltpu.sync_copy(z, out_hbm.at[c, pl.ds(r0, K)])
        return _
    lax.fori_loop(0, TW // K, _writeback, None)


@functools.partial(
    pl.kernel,
    out_type=jax.ShapeDtypeStruct((NC, NP, H), jnp.float32),
    mesh=_mesh,
    scratch_types=[
        pltpu.VMEM((K,), jnp.int32),
        pltpu.VMEM((K, H), jnp.float32),
        pltpu.VMEM_SHARED((NP, H), jnp.float32),
    ],
)
def _sc_degree(dst_hbm, out_hbm, dstk_v, rows_v, acc_sh):
    c = lax.axis_index("c")
    s = lax.axis_index("s")

    def _zero_row(r, _):
        for l in range(H // 16):
            rows_v[r, pl.ds(l * 16, 16)] = jnp.zeros((16,), jnp.float32)
        return _
    lax.fori_loop(0, K, _zero_row, None)

    def _zero_slice(j, _):
        pltpu.sync_copy(rows_v, acc_sh.at[pl.ds(s * TW + j * K, K)])
        return _
    lax.fori_loop(0, TW // K, _zero_slice, None)

    def _ones_row(r, _):
        for l in range(H // 16):
            rows_v[r, pl.ds(l * 16, 16)] = jnp.ones((16,), jnp.float32)
        return _
    lax.fori_loop(0, K, _ones_row, None)
    plsc.subcore_barrier()

    def _edge_chunk(i, _):
        pltpu.sync_copy(dst_hbm.at[pl.ds(c * ECP + s * EWP + i * K, K)], dstk_v)
        pltpu.sync_copy(rows_v, acc_sh.at[dstk_v], add=True)
        return _
    lax.fori_loop(0, NCHUNK, _edge_chunk, None)

    plsc.subcore_barrier()

    def _writeback(j, _):
        r0 = s * TW + j * K
        pltpu.sync_copy(acc_sh.at[pl.ds(r0, K)], rows_v)
        pltpu.sync_copy(rows_v, out_hbm.at[c, pl.ds(r0, K)])
        return _
    lax.fori_loop(0, TW // K, _writeback, None)


def _tc_deg_body(degt_ref, dinv_ref):
    deg = degt_ref[0, :, 0:1] + degt_ref[1, :, 0:1] + 1.0
    dinv_ref[...] = lax.rsqrt(deg)


def _tc_deg(degt):
    return pl.pallas_call(
        _tc_deg_body,
        grid=(GRID,),
        in_specs=[pl.BlockSpec((NC, BLK, H), lambda i: (0, i, 0))],
        out_specs=pl.BlockSpec((BLK, 1), lambda i: (i, 0)),
        out_shape=jax.ShapeDtypeStruct((NP, 1), jnp.float32),
    )(degt)


def _tc_a_body(x_ref, dinv_ref, w_ref, t_ref, hs_ref):
    t = jnp.dot(x_ref[...], w_ref[...], preferred_element_type=jnp.float32)
    t_ref[...] = t
    hs_ref[...] = t * dinv_ref[...]


def _tc_a(x_pad, dinv, W1):
    return pl.pallas_call(
        _tc_a_body,
        grid=(GRID,),
        in_specs=[
            pl.BlockSpec((BLK, D), lambda i: (i, 0)),
            pl.BlockSpec((BLK, 1), lambda i: (i, 0)),
            pl.BlockSpec((D, H), lambda i: (0, 0)),
        ],
        out_specs=[
            pl.BlockSpec((BLK, H), lambda i: (i, 0)),
            pl.BlockSpec((BLK, H), lambda i: (i, 0)),
        ],
        out_shape=[
            jax.ShapeDtypeStruct((NP, H), jnp.float32),
            jax.ShapeDtypeStruct((NP, H), jnp.float32),
        ],
    )(x_pad, dinv, W1)


def _tc_b_body(agg_ref, t_ref, dinv_ref, b_ref, w_ref, h_ref, t2_ref, hs2_ref):
    dinv = dinv_ref[...]
    out = dinv * (agg_ref[0] + agg_ref[1]) + dinv * dinv * t_ref[...] + b_ref[...]
    h = jnp.maximum(out, 0.0)
    h_ref[...] = h
    t2 = jnp.dot(h, w_ref[...], preferred_element_type=jnp.float32)
    t2_ref[...] = t2
    hs2_ref[...] = t2 * dinv


def _tc_b(agg1, t1, dinv, b1, W2):
    return pl.pallas_call(
        _tc_b_body,
        grid=(GRID,),
        in_specs=[
            pl.BlockSpec((NC, BLK, H), lambda i: (0, i, 0)),
            pl.BlockSpec((BLK, H), lambda i: (i, 0)),
            pl.BlockSpec((BLK, 1), lambda i: (i, 0)),
            pl.BlockSpec((1, H), lambda i: (0, 0)),
            pl.BlockSpec((H, H), lambda i: (0, 0)),
        ],
        out_specs=[
            pl.BlockSpec((BLK, H), lambda i: (i, 0)),
            pl.BlockSpec((BLK, H), lambda i: (i, 0)),
            pl.BlockSpec((BLK, H), lambda i: (i, 0)),
        ],
        out_shape=[
            jax.ShapeDtypeStruct((NP, H), jnp.float32),
            jax.ShapeDtypeStruct((NP, H), jnp.float32),
            jax.ShapeDtypeStruct((NP, H), jnp.float32),
        ],
    )(agg1, t1, dinv, b1, W2)


def _tc_c_body(agg_ref, t2_ref, h1_ref, dinv_ref, b_ref, w_ref, t3_ref, hs3_ref):
    dinv = dinv_ref[...]
    out = dinv * (agg_ref[0] + agg_ref[1]) + dinv * dinv * t2_ref[...] + b_ref[...]
    h2 = jnp.maximum(h1_ref[...] + out, 0.0)
    t3 = jnp.dot(h2, w_ref[...], preferred_element_type=jnp.float32)
    t3_ref[...] = t3
    hs3_ref[...] = t3 * dinv


def _tc_c(agg2, t2, h1, dinv, b2, W3):
    return pl.pallas_call(
        _tc_c_body,
        grid=(GRID,),
        in_specs=[
            pl.BlockSpec((NC, BLK, H), lambda i: (0, i, 0)),
            pl.BlockSpec((BLK, H), lambda i: (i, 0)),
            pl.BlockSpec((BLK, H), lambda i: (i, 0)),
            pl.BlockSpec((BLK, 1), lambda i: (i, 0)),
            pl.BlockSpec((1, H), lambda i: (0, 0)),
            pl.BlockSpec((H, H), lambda i: (0, 0)),
        ],
        out_specs=[
            pl.BlockSpec((BLK, H), lambda i: (i, 0)),
            pl.BlockSpec((BLK, H), lambda i: (i, 0)),
        ],
        out_shape=[
            jax.ShapeDtypeStruct((NP, H), jnp.float32),
            jax.ShapeDtypeStruct((NP, H), jnp.float32),
        ],
    )(agg2, t2, h1, dinv, b2, W3)


def _tc_d_body(agg_ref, t3_ref, dinv_ref, b_ref, batch_ref, wc_ref, bc_ref,
               logits_ref, logp_ref, sums_ref, cnt_ref):
    step = pl.program_id(0)

    @pl.when(step == 0)
    def _():
        sums_ref[...] = jnp.zeros_like(sums_ref)
        cnt_ref[...] = jnp.zeros_like(cnt_ref)

    dinv = dinv_ref[...]
    out3 = dinv * (agg_ref[0] + agg_ref[1]) + dinv * dinv * t3_ref[...] + b_ref[...]
    gids = lax.broadcasted_iota(jnp.int32, (1, G), 1)
    mask = (batch_ref[...] == gids).astype(jnp.float32)
    sums_ref[...] += lax.dot_general(
        mask, out3, (((0,), (0,)), ((), ())),
        preferred_element_type=jnp.float32)
    cnt_ref[...] += jnp.sum(mask, axis=0)[:, None]

    @pl.when(step == GRID - 1)
    def _():
        pooled = sums_ref[...] / jnp.maximum(cnt_ref[...], 1.0)
        logits = jnp.dot(pooled, wc_ref[...],
                         preferred_element_type=jnp.float32) + bc_ref[...]
        m = jnp.max(logits, axis=1, keepdims=True)
        lse = jnp.log(jnp.sum(jnp.exp(logits - m), axis=1, keepdims=True)) + m
        logits_ref[...] = logits
        logp_ref[...] = logits - lse


def _tc_d(agg3, t3, dinv, b3, batch_pad, Wc, bc):
    return pl.pallas_call(
        _tc_d_body,
        grid=(GRID,),
        in_specs=[
            pl.BlockSpec((NC, BLK, H), lambda i: (0, i, 0)),
            pl.BlockSpec((BLK, H), lambda i: (i, 0)),
            pl.BlockSpec((BLK, 1), lambda i: (i, 0)),
            pl.BlockSpec((1, H), lambda i: (0, 0)),
            pl.BlockSpec((BLK, 1), lambda i: (i, 0)),
            pl.BlockSpec((H, C), lambda i: (0, 0)),
            pl.BlockSpec((1, C), lambda i: (0, 0)),
        ],
        out_specs=[
            pl.BlockSpec((G, C), lambda i: (0, 0)),
            pl.BlockSpec((G, C), lambda i: (0, 0)),
        ],
        out_shape=[
            jax.ShapeDtypeStruct((G, C), jnp.float32),
            jax.ShapeDtypeStruct((G, C), jnp.float32),
        ],
        scratch_shapes=[
            pltpu.VMEM((G, H), jnp.float32),
            pltpu.VMEM((G, 1), jnp.float32),
        ],
    )(agg3, t3, dinv, b3, batch_pad, Wc, bc)


def kernel(x, edge_index, batch, W1, b1, W2, b2, W3, b3, Wc, bc):
    x_pad = jnp.pad(x, ((0, NP - N), (0, 0)))
    batch_pad = jnp.pad(batch, (0, NP - N), constant_values=-1).reshape(NP, 1)

    pad_idx = (N + jnp.arange(EP - E, dtype=jnp.int32) % (NP - N)).astype(jnp.int32)
    src = jnp.concatenate([edge_index[0], pad_idx])
    dst = jnp.concatenate([edge_index[1], pad_idx])

    degt = _sc_degree(dst)
    dinv = _tc_deg(degt)
    t1, hs1 = _tc_a(x_pad, dinv, W1)
    agg1 = _sc_scatter(hs1, src, dst)
    h1, t2, hs2 = _tc_b(agg1, t1, dinv, b1.reshape(1, H), W2)
    agg2 = _sc_scatter(hs2, src, dst)
    t3, hs3 = _tc_c(agg2, t2, h1, dinv, b2.reshape(1, H), W3)
    agg3 = _sc_scatter(hs3, src, dst)
    logits, logp = _tc_d(agg3, t3, dinv, b3.reshape(1, H), batch_pad,
                         Wc, bc.reshape(1, C))
    return (logits, logp)


# speedup vs baseline: 1.0918x; 1.0918x over previous
import functools

import jax
import jax.numpy as jnp
from jax import lax
from jax.experimental import pallas as pl
from jax.experimental.pallas import tpu as pltpu
from jax.experimental.pallas import tpu_sc as plsc

N = 10000
E = 320000
D = 128
H = 128
C = 10
G = 16

NC = 2
NS = 16
NW = NC * NS

NP = 10240
TW = NP // NS
K = 128
EP = 327680
EWP = EP // NW
ECP = EP // NC
NCHUNK = EWP // K

BLK = 1024
GRID = NP // BLK

_mesh = plsc.VectorSubcoreMesh(core_axis_name="c", subcore_axis_name="s",
                               num_cores=NC, num_subcores=NS)


@functools.partial(
    pl.kernel,
    out_type=jax.ShapeDtypeStruct((NC, NP, H), jnp.float32),
    mesh=_mesh,
    scratch_types=[
        pltpu.VMEM((EWP,), jnp.int32),
        pltpu.VMEM((K,), jnp.int32),
        pltpu.VMEM((K,), jnp.int32),
        pltpu.VMEM((2, K, H), jnp.float32),
        pltpu.VMEM_SHARED((NP, H), jnp.float32),
        pltpu.SemaphoreType.DMA,
        pltpu.SemaphoreType.DMA,
        pltpu.SemaphoreType.DMA,
        pltpu.SemaphoreType.DMA,
    ],
)
def _sc_scatter(hs_hbm, src_hbm, dst_hbm, out_hbm, src_v, dstk0_v, dstk1_v,
                rows_v, acc_sh, sem0, sem1, semd0, semd1):
    c = lax.axis_index("c")
    s = lax.axis_index("s")

    base0 = c * ECP + s * EWP
    pltpu.sync_copy(src_hbm.at[pl.ds(base0, EWP)], src_v)

    z = rows_v.at[0]

    def _zero_row(r, _):
        for l in range(H // 16):
            z[r, pl.ds(l * 16, 16)] = jnp.zeros((16,), jnp.float32)
        return _
    lax.fori_loop(0, K, _zero_row, None)

    def _zero_slice(j, _):
        pltpu.sync_copy(z, acc_sh.at[pl.ds(s * TW + j * K, K)])
        return _
    lax.fori_loop(0, TW // K, _zero_slice, None)
    plsc.subcore_barrier()

    def _issue(chunk, b, sem):
        pltpu.async_copy(hs_hbm.at[src_v.at[pl.ds(chunk * K, K)]], rows_v.at[b], sem)

    def _drain(b, sem):
        pltpu.make_async_copy(hs_hbm.at[pl.ds(0, K)], rows_v.at[b], sem).wait()

    def _issue_dst(chunk, dref, sem):
        pltpu.async_copy(dst_hbm.at[pl.ds(base0 + chunk * K, K)], dref, sem)

    def _drain_dst(dref, sem):
        pltpu.make_async_copy(dst_hbm.at[pl.ds(0, K)], dref, sem).wait()

    def _scatter(b, dref):
        pltpu.sync_copy(rows_v.at[b], acc_sh.at[dref], add=True)

    _issue(0, 0, sem0)
    _issue_dst(0, dstk0_v, semd0)

    def _edge_pair(g, _):
        c0 = 2 * g
        _issue(c0 + 1, 1, sem1)
        _issue_dst(c0 + 1, dstk1_v, semd1)
        _drain(0, sem0)
        _drain_dst(dstk0_v, semd0)
        _scatter(0, dstk0_v)
        nxt = c0 + 2
        nxt = jnp.where(nxt == NCHUNK, 0, nxt)
        _issue(nxt, 0, sem0)
        _issue_dst(nxt, dstk0_v, semd0)
        _drain(1, sem1)
        _drain_dst(dstk1_v, semd1)
        _scatter(1, dstk1_v)
        return _
    lax.fori_loop(0, NCHUNK // 2, _edge_pair, None)
    _drain(0, sem0)
    _drain_dst(dstk0_v, semd0)

    plsc.subcore_barrier()

    def _writeback(j, _):
        r0 = s * TW + j * K
        pltpu.sync_copy(acc_sh.at[pl.ds(r0, K)], z)
        pltpu.sync_copy(z, out_hbm.at[c, pl.ds(r0, K)])
        return _
    lax.fori_loop(0, TW // K, _writeback, None)


@functools.partial(
    pl.kernel,
    out_type=jax.ShapeDtypeStruct((NC, NP, H), jnp.float32),
    mesh=_mesh,
    scratch_types=[
        pltpu.VMEM((K,), jnp.int32),
        pltpu.VMEM((K, H), jnp.float32),
        pltpu.VMEM_SHARED((NP, H), jnp.float32),
    ],
)
def _sc_degree(dst_hbm, out_hbm, dstk_v, rows_v, acc_sh):
    c = lax.axis_index("c")
    s = lax.axis_index("s")

    def _zero_row(r, _):
        for l in range(H // 16):
            rows_v[r, pl.ds(l * 16, 16)] = jnp.zeros((16,), jnp.float32)
        return _
    lax.fori_loop(0, K, _zero_row, None)

    def _zero_slice(j, _):
        pltpu.sync_copy(rows_v, acc_sh.at[pl.ds(s * TW + j * K, K)])
        return _
    lax.fori_loop(0, TW // K, _zero_slice, None)

    def _ones_row(r, _):
        for l in range(H // 16):
            rows_v[r, pl.ds(l * 16, 16)] = jnp.ones((16,), jnp.float32)
        return _
    lax.fori_loop(0, K, _ones_row, None)
    plsc.subcore_barrier()

    def _edge_chunk(i, _):
        pltpu.sync_copy(dst_hbm.at[pl.ds(c * ECP + s * EWP + i * K, K)], dstk_v)
        pltpu.sync_copy(rows_v, acc_sh.at[dstk_v], add=True)
        return _
    lax.fori_loop(0, NCHUNK, _edge_chunk, None)

    plsc.subcore_barrier()

    def _writeback(j, _):
        r0 = s * TW + j * K
        pltpu.sync_copy(acc_sh.at[pl.ds(r0, K)], rows_v)
        pltpu.sync_copy(rows_v, out_hbm.at[c, pl.ds(r0, K)])
        return _
    lax.fori_loop(0, TW // K, _writeback, None)


def _tc_deg_body(degt_ref, dinv_ref):
    deg = degt_ref[0, :, 0:1] + degt_ref[1, :, 0:1] + 1.0
    dinv_ref[...] = lax.rsqrt(deg)


def _tc_deg(degt):
    return pl.pallas_call(
        _tc_deg_body,
        grid=(GRID,),
        in_specs=[pl.BlockSpec((NC, BLK, H), lambda i: (0, i, 0))],
        out_specs=pl.BlockSpec((BLK, 1), lambda i: (i, 0)),
        out_shape=jax.ShapeDtypeStruct((NP, 1), jnp.float32),
    )(degt)


def _tc_a_body(x_ref, dinv_ref, w_ref, t_ref, hs_ref):
    t = jnp.dot(x_ref[...], w_ref[...], preferred_element_type=jnp.float32)
    t_ref[...] = t
    hs_ref[...] = t * dinv_ref[...]


def _tc_a(x_pad, dinv, W1):
    return pl.pallas_call(
        _tc_a_body,
        grid=(GRID,),
        in_specs=[
            pl.BlockSpec((BLK, D), lambda i: (i, 0)),
            pl.BlockSpec((BLK, 1), lambda i: (i, 0)),
            pl.BlockSpec((D, H), lambda i: (0, 0)),
        ],
        out_specs=[
            pl.BlockSpec((BLK, H), lambda i: (i, 0)),
            pl.BlockSpec((BLK, H), lambda i: (i, 0)),
        ],
        out_shape=[
            jax.ShapeDtypeStruct((NP, H), jnp.float32),
            jax.ShapeDtypeStruct((NP, H), jnp.float32),
        ],
    )(x_pad, dinv, W1)


def _tc_b_body(agg_ref, t_ref, dinv_ref, b_ref, w_ref, h_ref, t2_ref, hs2_ref):
    dinv = dinv_ref[...]
    out = dinv * (agg_ref[0] + agg_ref[1]) + dinv * dinv * t_ref[...] + b_ref[...]
    h = jnp.maximum(out, 0.0)
    h_ref[...] = h
    t2 = jnp.dot(h, w_ref[...], preferred_element_type=jnp.float32)
    t2_ref[...] = t2
    hs2_ref[...] = t2 * dinv


def _tc_b(agg1, t1, dinv, b1, W2):
    return pl.pallas_call(
        _tc_b_body,
        grid=(GRID,),
        in_specs=[
            pl.BlockSpec((NC, BLK, H), lambda i: (0, i, 0)),
            pl.BlockSpec((BLK, H), lambda i: (i, 0)),
            pl.BlockSpec((BLK, 1), lambda i: (i, 0)),
            pl.BlockSpec((1, H), lambda i: (0, 0)),
            pl.BlockSpec((H, H), lambda i: (0, 0)),
        ],
        out_specs=[
            pl.BlockSpec((BLK, H), lambda i: (i, 0)),
            pl.BlockSpec((BLK, H), lambda i: (i, 0)),
            pl.BlockSpec((BLK, H), lambda i: (i, 0)),
        ],
        out_shape=[
            jax.ShapeDtypeStruct((NP, H), jnp.float32),
            jax.ShapeDtypeStruct((NP, H), jnp.float32),
            jax.ShapeDtypeStruct((NP, H), jnp.float32),
        ],
    )(agg1, t1, dinv, b1, W2)


def _tc_c_body(agg_ref, t2_ref, h1_ref, dinv_ref, b_ref, w_ref, t3_ref, hs3_ref):
    dinv = dinv_ref[...]
    out = dinv * (agg_ref[0] + agg_ref[1]) + dinv * dinv * t2_ref[...] + b_ref[...]
    h2 = jnp.maximum(h1_ref[...] + out, 0.0)
    t3 = jnp.dot(h2, w_ref[...], preferred_element_type=jnp.float32)
    t3_ref[...] = t3
    hs3_ref[...] = t3 * dinv


def _tc_c(agg2, t2, h1, dinv, b2, W3):
    return pl.pallas_call(
        _tc_c_body,
        grid=(GRID,),
        in_specs=[
            pl.BlockSpec((NC, BLK, H), lambda i: (0, i, 0)),
            pl.BlockSpec((BLK, H), lambda i: (i, 0)),
            pl.BlockSpec((BLK, H), lambda i: (i, 0)),
            pl.BlockSpec((BLK, 1), lambda i: (i, 0)),
            pl.BlockSpec((1, H), lambda i: (0, 0)),
            pl.BlockSpec((H, H), lambda i: (0, 0)),
        ],
        out_specs=[
            pl.BlockSpec((BLK, H), lambda i: (i, 0)),
            pl.BlockSpec((BLK, H), lambda i: (i, 0)),
        ],
        out_shape=[
            jax.ShapeDtypeStruct((NP, H), jnp.float32),
            jax.ShapeDtypeStruct((NP, H), jnp.float32),
        ],
    )(agg2, t2, h1, dinv, b2, W3)


def _tc_d_body(agg_ref, t3_ref, dinv_ref, b_ref, batch_ref, wc_ref, bc_ref,
               logits_ref, logp_ref, sums_ref, cnt_ref):
    step = pl.program_id(0)

    @pl.when(step == 0)
    def _():
        sums_ref[...] = jnp.zeros_like(sums_ref)
        cnt_ref[...] = jnp.zeros_like(cnt_ref)

    dinv = dinv_ref[...]
    out3 = dinv * (agg_ref[0] + agg_ref[1]) + dinv * dinv * t3_ref[...] + b_ref[...]
    gids = lax.broadcasted_iota(jnp.int32, (1, G), 1)
    mask = (batch_ref[...] == gids).astype(jnp.float32)
    sums_ref[...] += lax.dot_general(
        mask, out3, (((0,), (0,)), ((), ())),
        preferred_element_type=jnp.float32)
    cnt_ref[...] += jnp.sum(mask, axis=0)[:, None]

    @pl.when(step == GRID - 1)
    def _():
        pooled = sums_ref[...] / jnp.maximum(cnt_ref[...], 1.0)
        logits = jnp.dot(pooled, wc_ref[...],
                         preferred_element_type=jnp.float32) + bc_ref[...]
        m = jnp.max(logits, axis=1, keepdims=True)
        lse = jnp.log(jnp.sum(jnp.exp(logits - m), axis=1, keepdims=True)) + m
        logits_ref[...] = logits
        logp_ref[...] = logits - lse


def _tc_d(agg3, t3, dinv, b3, batch_pad, Wc, bc):
    return pl.pallas_call(
        _tc_d_body,
        grid=(GRID,),
        in_specs=[
            pl.BlockSpec((NC, BLK, H), lambda i: (0, i, 0)),
            pl.BlockSpec((BLK, H), lambda i: (i, 0)),
            pl.BlockSpec((BLK, 1), lambda i: (i, 0)),
            pl.BlockSpec((1, H), lambda i: (0, 0)),
            pl.BlockSpec((BLK, 1), lambda i: (i, 0)),
            pl.BlockSpec((H, C), lambda i: (0, 0)),
            pl.BlockSpec((1, C), lambda i: (0, 0)),
        ],
        out_specs=[
            pl.BlockSpec((G, C), lambda i: (0, 0)),
            pl.BlockSpec((G, C), lambda i: (0, 0)),
        ],
        out_shape=[
            jax.ShapeDtypeStruct((G, C), jnp.float32),
            jax.ShapeDtypeStruct((G, C), jnp.float32),
        ],
        scratch_shapes=[
            pltpu.VMEM((G, H), jnp.float32),
            pltpu.VMEM((G, 1), jnp.float32),
        ],
    )(agg3, t3, dinv, b3, batch_pad, Wc, bc)


def kernel(x, edge_index, batch, W1, b1, W2, b2, W3, b3, Wc, bc):
    x_pad = jnp.pad(x, ((0, NP - N), (0, 0)))
    batch_pad = jnp.pad(batch, (0, NP - N), constant_values=-1).reshape(NP, 1)

    pad_idx = (N + jnp.arange(EP - E, dtype=jnp.int32) % (NP - N)).astype(jnp.int32)
    src = jnp.concatenate([edge_index[0], pad_idx])
    dst = jnp.concatenate([edge_index[1], pad_idx])

    degt = _sc_degree(dst)
    dinv = _tc_deg(degt)
    t1, hs1 = _tc_a(x_pad, dinv, W1)
    agg1 = _sc_scatter(hs1, src, dst)
    h1, t2, hs2 = _tc_b(agg1, t1, dinv, b1.reshape(1, H), W2)
    agg2 = _sc_scatter(hs2, src, dst)
    t3, hs3 = _tc_c(agg2, t2, h1, dinv, b2.reshape(1, H), W3)
    agg3 = _sc_scatter(hs3, src, dst)
    logits, logp = _tc_d(agg3, t3, dinv, b3.reshape(1, H), batch_pad,
                         Wc, bc.reshape(1, C))
    return (logits, logp)


# speedup vs baseline: 1.1659x; 1.0679x over previous
import functools

import jax
import jax.numpy as jnp
from jax import lax
from jax.experimental import pallas as pl
from jax.experimental.pallas import tpu as pltpu
from jax.experimental.pallas import tpu_sc as plsc

N = 10000
E = 320000
D = 128
H = 128
C = 10
G = 16

NC = 2
NS = 16
NW = NC * NS

NP = 10240
TW = NP // NS
K = 128
EP = 327680
EWP = EP // NW
ECP = EP // NC
NCHUNK = EWP // K

BLK = 1024
GRID = NP // BLK

_mesh = plsc.VectorSubcoreMesh(core_axis_name="c", subcore_axis_name="s",
                               num_cores=NC, num_subcores=NS)


@functools.partial(
    pl.kernel,
    out_type=jax.ShapeDtypeStruct((NC, NP, H), jnp.float32),
    mesh=_mesh,
    scratch_types=[
        pltpu.VMEM((EWP,), jnp.int32),
        pltpu.VMEM((K,), jnp.int32),
        pltpu.VMEM((K,), jnp.int32),
        pltpu.VMEM((2, K, H), jnp.float32),
        pltpu.VMEM_SHARED((NP, H), jnp.float32),
        pltpu.SemaphoreType.DMA,
        pltpu.SemaphoreType.DMA,
        pltpu.SemaphoreType.DMA,
        pltpu.SemaphoreType.DMA,
    ],
)
def _sc_scatter(hs_hbm, src_hbm, dst_hbm, out_hbm, src_v, dstk0_v, dstk1_v,
                rows_v, acc_sh, sem0, sem1, semd0, semd1):
    c = lax.axis_index("c")
    s = lax.axis_index("s")

    base0 = c * ECP + s * EWP
    pltpu.sync_copy(src_hbm.at[pl.ds(base0, EWP)], src_v)

    z = rows_v.at[0]

    def _zero_row(r, _):
        for l in range(H // 16):
            z[r, pl.ds(l * 16, 16)] = jnp.zeros((16,), jnp.float32)
        return _
    lax.fori_loop(0, K, _zero_row, None)

    def _zero_slice(j, _):
        pltpu.sync_copy(z, acc_sh.at[pl.ds(s * TW + j * K, K)])
        return _
    lax.fori_loop(0, TW // K, _zero_slice, None)
    plsc.subcore_barrier()

    def _issue(chunk, b, sem):
        pltpu.async_copy(hs_hbm.at[src_v.at[pl.ds(chunk * K, K)]], rows_v.at[b], sem)

    def _drain(b, sem):
        pltpu.make_async_copy(hs_hbm.at[pl.ds(0, K)], rows_v.at[b], sem).wait()

    def _issue_dst(chunk, dref, sem):
        pltpu.async_copy(dst_hbm.at[pl.ds(base0 + chunk * K, K)], dref, sem)

    def _drain_dst(dref, sem):
        pltpu.make_async_copy(dst_hbm.at[pl.ds(0, K)], dref, sem).wait()

    def _scatter(b, dref):
        pltpu.sync_copy(rows_v.at[b], acc_sh.at[dref], add=True)

    _issue(0, 0, sem0)
    _issue_dst(0, dstk0_v, semd0)

    def _edge_pair(g, _):
        c0 = 2 * g
        _issue(c0 + 1, 1, sem1)
        _issue_dst(c0 + 1, dstk1_v, semd1)
        _drain(0, sem0)
        _drain_dst(dstk0_v, semd0)
        _scatter(0, dstk0_v)
        nxt = c0 + 2
        nxt = jnp.where(nxt == NCHUNK, 0, nxt)
        _issue(nxt, 0, sem0)
        _issue_dst(nxt, dstk0_v, semd0)
        _drain(1, sem1)
        _drain_dst(dstk1_v, semd1)
        _scatter(1, dstk1_v)
        return _
    lax.fori_loop(0, NCHUNK // 2, _edge_pair, None)
    _drain(0, sem0)
    _drain_dst(dstk0_v, semd0)

    plsc.subcore_barrier()

    def _writeback(j, _):
        r0 = s * TW + j * K
        pltpu.sync_copy(acc_sh.at[pl.ds(r0, K)], z)
        pltpu.sync_copy(z, out_hbm.at[c, pl.ds(r0, K)])
        return _
    lax.fori_loop(0, TW // K, _writeback, None)


@functools.partial(
    pl.kernel,
    out_type=jax.ShapeDtypeStruct((NC, NP, H), jnp.float32),
    mesh=_mesh,
    scratch_types=[
        pltpu.VMEM((K,), jnp.int32),
        pltpu.VMEM((K,), jnp.int32),
        pltpu.VMEM((K, H), jnp.float32),
        pltpu.VMEM_SHARED((NP, H), jnp.float32),
        pltpu.SemaphoreType.DMA,
        pltpu.SemaphoreType.DMA,
    ],
)
def _sc_degree(dst_hbm, out_hbm, dstk0_v, dstk1_v, rows_v, acc_sh, semd0, semd1):
    c = lax.axis_index("c")
    s = lax.axis_index("s")
    base0 = c * ECP + s * EWP

    def _zero_row(r, _):
        for l in range(H // 16):
            rows_v[r, pl.ds(l * 16, 16)] = jnp.zeros((16,), jnp.float32)
        return _
    lax.fori_loop(0, K, _zero_row, None)

    def _zero_slice(j, _):
        pltpu.sync_copy(rows_v, acc_sh.at[pl.ds(s * TW + j * K, K)])
        return _
    lax.fori_loop(0, TW // K, _zero_slice, None)

    def _ones_row(r, _):
        for l in range(H // 16):
            rows_v[r, pl.ds(l * 16, 16)] = jnp.ones((16,), jnp.float32)
        return _
    lax.fori_loop(0, K, _ones_row, None)
    plsc.subcore_barrier()

    def _issue_dst(chunk, dref, sem):
        pltpu.async_copy(dst_hbm.at[pl.ds(base0 + chunk * K, K)], dref, sem)

    def _drain_dst(dref, sem):
        pltpu.make_async_copy(dst_hbm.at[pl.ds(0, K)], dref, sem).wait()

    _issue_dst(0, dstk0_v, semd0)

    def _edge_pair(g, _):
        c0 = 2 * g
        _issue_dst(c0 + 1, dstk1_v, semd1)
        _drain_dst(dstk0_v, semd0)
        pltpu.sync_copy(rows_v, acc_sh.at[dstk0_v], add=True)
        nxt = c0 + 2
        nxt = jnp.where(nxt == NCHUNK, 0, nxt)
        _issue_dst(nxt, dstk0_v, semd0)
        _drain_dst(dstk1_v, semd1)
        pltpu.sync_copy(rows_v, acc_sh.at[dstk1_v], add=True)
        return _
    lax.fori_loop(0, NCHUNK // 2, _edge_pair, None)
    _drain_dst(dstk0_v, semd0)

    plsc.subcore_barrier()

    def _writeback(j, _):
        r0 = s * TW + j * K
        pltpu.sync_copy(acc_sh.at[pl.ds(r0, K)], rows_v)
        pltpu.sync_copy(rows_v, out_hbm.at[c, pl.ds(r0, K)])
        return _
    lax.fori_loop(0, TW // K, _writeback, None)


def _tc_deg_body(degt_ref, dinv_ref):
    deg = degt_ref[0, :, 0:1] + degt_ref[1, :, 0:1] + 1.0
    dinv_ref[...] = lax.rsqrt(deg)


def _tc_deg(degt):
    return pl.pallas_call(
        _tc_deg_body,
        grid=(GRID,),
        in_specs=[pl.BlockSpec((NC, BLK, H), lambda i: (0, i, 0))],
        out_specs=pl.BlockSpec((BLK, 1), lambda i: (i, 0)),
        out_shape=jax.ShapeDtypeStruct((NP, 1), jnp.float32),
    )(degt)


def _tc_a_body(x_ref, dinv_ref, w_ref, t_ref, hs_ref):
    t = jnp.dot(x_ref[...], w_ref[...], preferred_element_type=jnp.float32)
    t_ref[...] = t
    hs_ref[...] = t * dinv_ref[...]


def _tc_a(x_pad, dinv, W1):
    return pl.pallas_call(
        _tc_a_body,
        grid=(GRID,),
        in_specs=[
            pl.BlockSpec((BLK, D), lambda i: (i, 0)),
            pl.BlockSpec((BLK, 1), lambda i: (i, 0)),
            pl.BlockSpec((D, H), lambda i: (0, 0)),
        ],
        out_specs=[
            pl.BlockSpec((BLK, H), lambda i: (i, 0)),
            pl.BlockSpec((BLK, H), lambda i: (i, 0)),
        ],
        out_shape=[
            jax.ShapeDtypeStruct((NP, H), jnp.float32),
            jax.ShapeDtypeStruct((NP, H), jnp.float32),
        ],
    )(x_pad, dinv, W1)


def _tc_b_body(agg_ref, t_ref, dinv_ref, b_ref, w_ref, h_ref, t2_ref, hs2_ref):
    dinv = dinv_ref[...]
    out = dinv * (agg_ref[0] + agg_ref[1]) + dinv * dinv * t_ref[...] + b_ref[...]
    h = jnp.maximum(out, 0.0)
    h_ref[...] = h
    t2 = jnp.dot(h, w_ref[...], preferred_element_type=jnp.float32)
    t2_ref[...] = t2
    hs2_ref[...] = t2 * dinv


def _tc_b(agg1, t1, dinv, b1, W2):
    return pl.pallas_call(
        _tc_b_body,
        grid=(GRID,),
        in_specs=[
            pl.BlockSpec((NC, BLK, H), lambda i: (0, i, 0)),
            pl.BlockSpec((BLK, H), lambda i: (i, 0)),
            pl.BlockSpec((BLK, 1), lambda i: (i, 0)),
            pl.BlockSpec((1, H), lambda i: (0, 0)),
            pl.BlockSpec((H, H), lambda i: (0, 0)),
        ],
        out_specs=[
            pl.BlockSpec((BLK, H), lambda i: (i, 0)),
            pl.BlockSpec((BLK, H), lambda i: (i, 0)),
            pl.BlockSpec((BLK, H), lambda i: (i, 0)),
        ],
        out_shape=[
            jax.ShapeDtypeStruct((NP, H), jnp.float32),
            jax.ShapeDtypeStruct((NP, H), jnp.float32),
            jax.ShapeDtypeStruct((NP, H), jnp.float32),
        ],
    )(agg1, t1, dinv, b1, W2)


def _tc_c_body(agg_ref, t2_ref, h1_ref, dinv_ref, b_ref, w_ref, t3_ref, hs3_ref):
    dinv = dinv_ref[...]
    out = dinv * (agg_ref[0] + agg_ref[1]) + dinv * dinv * t2_ref[...] + b_ref[...]
    h2 = jnp.maximum(h1_ref[...] + out, 0.0)
    t3 = jnp.dot(h2, w_ref[...], preferred_element_type=jnp.float32)
    t3_ref[...] = t3
    hs3_ref[...] = t3 * dinv


def _tc_c(agg2, t2, h1, dinv, b2, W3):
    return pl.pallas_call(
        _tc_c_body,
        grid=(GRID,),
        in_specs=[
            pl.BlockSpec((NC, BLK, H), lambda i: (0, i, 0)),
            pl.BlockSpec((BLK, H), lambda i: (i, 0)),
            pl.BlockSpec((BLK, H), lambda i: (i, 0)),
            pl.BlockSpec((BLK, 1), lambda i: (i, 0)),
            pl.BlockSpec((1, H), lambda i: (0, 0)),
            pl.BlockSpec((H, H), lambda i: (0, 0)),
        ],
        out_specs=[
            pl.BlockSpec((BLK, H), lambda i: (i, 0)),
            pl.BlockSpec((BLK, H), lambda i: (i, 0)),
        ],
        out_shape=[
            jax.ShapeDtypeStruct((NP, H), jnp.float32),
            jax.ShapeDtypeStruct((NP, H), jnp.float32),
        ],
    )(agg2, t2, h1, dinv, b2, W3)


def _tc_d_body(agg_ref, t3_ref, dinv_ref, b_ref, batch_ref, wc_ref, bc_ref,
               logits_ref, logp_ref, sums_ref, cnt_ref):
    step = pl.program_id(0)

    @pl.when(step == 0)
    def _():
        sums_ref[...] = jnp.zeros_like(sums_ref)
        cnt_ref[...] = jnp.zeros_like(cnt_ref)

    dinv = dinv_ref[...]
    out3 = dinv * (agg_ref[0] + agg_ref[1]) + dinv * dinv * t3_ref[...] + b_ref[...]
    gids = lax.broadcasted_iota(jnp.int32, (1, G), 1)
    mask = (batch_ref[...] == gids).astype(jnp.float32)
    sums_ref[...] += lax.dot_general(
        mask, out3, (((0,), (0,)), ((), ())),
        preferred_element_type=jnp.float32)
    cnt_ref[...] += jnp.sum(mask, axis=0)[:, None]

    @pl.when(step == GRID - 1)
    def _():
        pooled = sums_ref[...] / jnp.maximum(cnt_ref[...], 1.0)
        logits = jnp.dot(pooled, wc_ref[...],
                         preferred_element_type=jnp.float32) + bc_ref[...]
        m = jnp.max(logits, axis=1, keepdims=True)
        lse = jnp.log(jnp.sum(jnp.exp(logits - m), axis=1, keepdims=True)) + m
        logits_ref[...] = logits
        logp_ref[...] = logits - lse


def _tc_d(agg3, t3, dinv, b3, batch_pad, Wc, bc):
    return pl.pallas_call(
        _tc_d_body,
        grid=(GRID,),
        in_specs=[
            pl.BlockSpec((NC, BLK, H), lambda i: (0, i, 0)),
            pl.BlockSpec((BLK, H), lambda i: (i, 0)),
            pl.BlockSpec((BLK, 1), lambda i: (i, 0)),
            pl.BlockSpec((1, H), lambda i: (0, 0)),
            pl.BlockSpec((BLK, 1), lambda i: (i, 0)),
            pl.BlockSpec((H, C), lambda i: (0, 0)),
            pl.BlockSpec((1, C), lambda i: (0, 0)),
        ],
        out_specs=[
            pl.BlockSpec((G, C), lambda i: (0, 0)),
            pl.BlockSpec((G, C), lambda i: (0, 0)),
        ],
        out_shape=[
            jax.ShapeDtypeStruct((G, C), jnp.float32),
            jax.ShapeDtypeStruct((G, C), jnp.float32),
        ],
        scratch_shapes=[
            pltpu.VMEM((G, H), jnp.float32),
            pltpu.VMEM((G, 1), jnp.float32),
        ],
    )(agg3, t3, dinv, b3, batch_pad, Wc, bc)


def kernel(x, edge_index, batch, W1, b1, W2, b2, W3, b3, Wc, bc):
    x_pad = jnp.pad(x, ((0, NP - N), (0, 0)))
    batch_pad = jnp.pad(batch, (0, NP - N), constant_values=-1).reshape(NP, 1)

    pad_idx = (N + jnp.arange(EP - E, dtype=jnp.int32) % (NP - N)).astype(jnp.int32)
    src = jnp.concatenate([edge_index[0], pad_idx])
    dst = jnp.concatenate([edge_index[1], pad_idx])

    degt = _sc_degree(dst)
    dinv = _tc_deg(degt)
    t1, hs1 = _tc_a(x_pad, dinv, W1)
    agg1 = _sc_scatter(hs1, src, dst)
    h1, t2, hs2 = _tc_b(agg1, t1, dinv, b1.reshape(1, H), W2)
    agg2 = _sc_scatter(hs2, src, dst)
    t3, hs3 = _tc_c(agg2, t2, h1, dinv, b2.reshape(1, H), W3)
    agg3 = _sc_scatter(hs3, src, dst)
    logits, logp = _tc_d(agg3, t3, dinv, b3.reshape(1, H), batch_pad,
                         Wc, bc.reshape(1, C))
    return (logits, logp)
